# baseline (speedup 1.0000x reference)
import functools

import numpy as np
import jax
import jax.numpy as jnp
from jax import lax
from jax.experimental import pallas as pl
from jax.experimental.pallas import tpu as pltpu
from jax.experimental.pallas import tpu_sc as plsc

_CUT_WIDTH = 64
_STRIPES_NUM = 2

_NC = 2
_NS = 16
_NW = _NC * _NS

_CH = 128
_NBUF = 3


@functools.lru_cache(maxsize=None)
def _gather_rows(batch: int, width: int) -> np.ndarray:
    rng = np.random.default_rng(0)
    perm = rng.permutation(batch)
    src = np.tile(np.arange(batch, dtype=np.int64)[:, None], (1, width))
    for n in range(batch):
        for _ in range(_STRIPES_NUM):
            distance = int(rng.integers(0, _CUT_WIDTH))
            bgn = int(rng.integers(0, width - distance))
            if distance:
                src[n, bgn:bgn + distance] = perm[n]
    rows = src * width + np.arange(width)[None, :]
    return rows.reshape(-1).astype(np.int32)


def _sc_gather(x, idx, rows, feat, n_chunk):
    mesh = plsc.VectorSubcoreMesh(core_axis_name="c", subcore_axis_name="s")

    @functools.partial(
        pl.kernel,
        out_type=jax.ShapeDtypeStruct((rows, feat), jnp.float32),
        mesh=mesh,
        scratch_types=[
            pltpu.VMEM((2 * n_chunk, _CH), jnp.int32),
        ] + [pltpu.VMEM((2 * _CH, feat), jnp.float32)] * _NBUF
          + [pltpu.SemaphoreType.DMA] * (2 * _NBUF),
    )
    def body(x_hbm, idx_hbm, out_hbm, idx_v, *bufs_sems):
        bufs = bufs_sems[:_NBUF]
        gsems = bufs_sems[_NBUF:2 * _NBUF]
        wsems = bufs_sems[2 * _NBUF:]
        wid = lax.axis_index("s") * _NC + lax.axis_index("c")
        base = wid * (n_chunk * 2 * _CH)
        pltpu.sync_copy(idx_hbm.at[wid], idx_v)

        def gather(c):
            b = c % _NBUF
            h0 = pltpu.async_copy(
                x_hbm.at[idx_v.at[2 * c]],
                bufs[b].at[pl.ds(0, _CH)], gsems[b])
            h1 = pltpu.async_copy(
                x_hbm.at[idx_v.at[2 * c + 1]],
                bufs[b].at[pl.ds(_CH, _CH)], gsems[b])
            return (h0, h1)

        gh = {c: gather(c) for c in range(min(_NBUF, n_chunk))}
        wh = {}
        for c in range(n_chunk):
            if c > 0 and c - 1 + _NBUF < n_chunk:
                wh[c - 1].wait()
                gh[c - 1 + _NBUF] = gather(c - 1 + _NBUF)
            gh[c][0].wait()
            gh[c][1].wait()
            wh[c] = pltpu.async_copy(
                bufs[c % _NBUF],
                out_hbm.at[pl.ds(base + c * (2 * _CH), 2 * _CH)],
                wsems[c % _NBUF])
        for c in range(max(0, n_chunk - _NBUF), n_chunk):
            wh[c].wait()

    return body(x, idx)


def kernel(input):
    batch, chan, width, feat = input.shape
    rows = batch * chan * width
    per_w = rows // _NW
    n_chunk = per_w // (2 * _CH)
    g = _gather_rows(batch, width).reshape(_NW, 2 * n_chunk, _CH)
    x = input.reshape(rows, feat)
    out = _sc_gather(x, jnp.asarray(g), rows, feat, n_chunk)
    return out.reshape(input.shape)
